# Initial kernel scaffold; baseline (speedup 1.0000x reference)
import jax
import jax.numpy as jnp
from jax.experimental import pallas as pl


def kernel(emojis, table):
    raise NotImplementedError("write your pallas kernel here")


# speedup vs baseline: 15.5241x; 15.5241x over previous
import jax
import jax.numpy as jnp
from jax import lax
from jax.experimental import pallas as pl
from jax.experimental.pallas import tpu as pltpu


_TILE = 8192
_UNROLL = 64
_VMEM_LIMIT = 48 << 20


def _gather_kernel(ids_ref, table_ref, out_ref, ids_smem, sem):
    tile = ids_ref.shape[-1]
    cp = pltpu.make_async_copy(ids_ref.at[0, 0], ids_smem, sem)
    cp.start()
    cp.wait()

    def body(j, carry):
        base = j * _UNROLL
        for mi in range(_UNROLL):
            idx = ids_smem[base + mi]
            out_ref[base + mi, 0] = table_ref[idx, 0]
        return carry

    lax.fori_loop(0, tile // _UNROLL, body, 0)


@jax.jit
def _emoji_gather(emojis, table):
    B, S = emojis.shape
    V, D = table.shape
    n = B * S

    ids = emojis.reshape(-1).astype(jnp.int32)
    tile = _TILE if n >= _TILE else max(_UNROLL, pl.cdiv(n, _UNROLL) * _UNROLL)
    n_pad = pl.cdiv(n, tile) * tile
    n_tiles = n_pad // tile
    if n_pad != n:
        ids = jnp.pad(ids, (0, n_pad - n))
    ids3 = ids.reshape(n_tiles, 1, tile)
    table3 = table.astype(jnp.float32).reshape(V, 1, D)

    out = pl.pallas_call(
        _gather_kernel,
        out_shape=jax.ShapeDtypeStruct((n_pad, 1, D), jnp.float32),
        grid=(n_tiles,),
        in_specs=[
            pl.BlockSpec((1, 1, tile), lambda i: (i, 0, 0)),
            pl.BlockSpec((V, 1, D), lambda i: (0, 0, 0),
                         pipeline_mode=pl.Buffered(1)),
        ],
        out_specs=pl.BlockSpec((tile, 1, D), lambda i: (i, 0, 0)),
        scratch_shapes=[
            pltpu.SMEM((tile,), jnp.int32),
            pltpu.SemaphoreType.DMA,
        ],
        compiler_params=pltpu.CompilerParams(
            dimension_semantics=("parallel",),
            vmem_limit_bytes=_VMEM_LIMIT,
        ),
    )(ids3, table3)

    if n_pad != n:
        out = out[:n]
    return out.reshape(B, S, D)


def kernel(emojis, table):
    return _emoji_gather(emojis, table)


# speedup vs baseline: 29.0912x; 1.8739x over previous
import jax
import jax.numpy as jnp
from jax import lax
from jax.experimental import pallas as pl
from jax.experimental.pallas import tpu as pltpu
from jax.sharding import PartitionSpec as P


_TILE = 8192
_UNROLL = 64
_VMEM_LIMIT = 48 << 20


def _gather_kernel(ids_ref, table_ref, out_ref, ids_smem, sem):
    tile = ids_ref.shape[-1]
    cp = pltpu.make_async_copy(ids_ref.at[0, 0], ids_smem, sem)
    cp.start()
    cp.wait()

    def body(j, carry):
        base = j * _UNROLL
        for mi in range(_UNROLL):
            idx = ids_smem[base + mi]
            out_ref[base + mi, 0] = table_ref[idx, 0]
        return carry

    lax.fori_loop(0, tile // _UNROLL, body, 0)


def _gather_tiles(ids3, table3):
    n_tiles, _, tile = ids3.shape
    V, _, D = table3.shape
    return pl.pallas_call(
        _gather_kernel,
        out_shape=jax.ShapeDtypeStruct((n_tiles * tile, 1, D), jnp.float32),
        grid=(n_tiles,),
        in_specs=[
            pl.BlockSpec((1, 1, tile), lambda i: (i, 0, 0)),
            pl.BlockSpec((V, 1, D), lambda i: (0, 0, 0),
                         pipeline_mode=pl.Buffered(1)),
        ],
        out_specs=pl.BlockSpec((tile, 1, D), lambda i: (i, 0, 0)),
        scratch_shapes=[
            pltpu.SMEM((tile,), jnp.int32),
            pltpu.SemaphoreType.DMA,
        ],
        compiler_params=pltpu.CompilerParams(
            dimension_semantics=("arbitrary",),
            vmem_limit_bytes=_VMEM_LIMIT,
        ),
    )(ids3, table3)


@jax.jit
def _emoji_gather(emojis, table):
    B, S = emojis.shape
    V, D = table.shape
    n = B * S

    ids = emojis.reshape(-1).astype(jnp.int32)
    tile = _TILE if n >= _TILE else max(_UNROLL, pl.cdiv(n, _UNROLL) * _UNROLL)
    n_pad = pl.cdiv(n, tile) * tile
    n_tiles = n_pad // tile
    if n_pad != n:
        ids = jnp.pad(ids, (0, n_pad - n))
    ids3 = ids.reshape(n_tiles, 1, tile)
    table3 = table.astype(jnp.float32).reshape(V, 1, D)

    ndev = len(jax.devices())
    if ndev > 1 and n_pad == n and n_tiles % ndev == 0 and B % ndev == 0:
        mesh = jax.make_mesh(
            (ndev,), ("c",),
            axis_types=(jax.sharding.AxisType.Explicit,),
        )
        ids3 = jax.reshard(ids3, jax.NamedSharding(mesh, P("c", None, None)))
        table3 = jax.reshard(table3, jax.NamedSharding(mesh, P()))

        def _local(i3, t3):
            return _gather_tiles(i3, t3).reshape(B // ndev, S, D)

        return jax.shard_map(
            _local, mesh=mesh, in_specs=(P("c"), P()), out_specs=P("c"),
            check_vma=False,
        )(ids3, table3)

    out = _gather_tiles(ids3, table3)
    if n_pad != n:
        out = out[:n]
    return out.reshape(B, S, D)


def kernel(emojis, table):
    return _emoji_gather(emojis, table)


# speedup vs baseline: 31.0635x; 1.0678x over previous
import jax
import jax.numpy as jnp
from jax import lax
from jax.experimental import pallas as pl
from jax.experimental.pallas import tpu as pltpu
from jax.sharding import PartitionSpec as P


_TILE = 8192
_UNROLL = 512
_CHUNK = 2048
_VMEM_LIMIT = 48 << 20


def _gather_kernel(ids_ref, table_ref, out_ref, ids_smem, sems):
    tile = ids_ref.shape[-1]
    n_chunk = tile // _CHUNK

    def _copy(c):
        return pltpu.make_async_copy(
            ids_ref.at[0, 0, pl.ds(c * _CHUNK, _CHUNK)],
            ids_smem.at[pl.ds(c * _CHUNK, _CHUNK)],
            sems.at[c],
        )

    _copy(0).start()
    for c in range(n_chunk):
        if c + 1 < n_chunk:
            _copy(c + 1).start()
        _copy(c).wait()

        def body(j, carry, c=c):
            base = c * _CHUNK + j * _UNROLL
            for mi in range(_UNROLL):
                idx = ids_smem[base + mi]
                out_ref[base + mi, 0] = table_ref[idx, 0]
            return carry

        lax.fori_loop(0, _CHUNK // _UNROLL, body, 0)


def _gather_tiles(ids3, table3):
    n_tiles, _, tile = ids3.shape
    V, _, D = table3.shape
    return pl.pallas_call(
        _gather_kernel,
        out_shape=jax.ShapeDtypeStruct((n_tiles * tile, 1, D), jnp.float32),
        grid=(n_tiles,),
        in_specs=[
            pl.BlockSpec((1, 1, tile), lambda i: (i, 0, 0)),
            pl.BlockSpec((V, 1, D), lambda i: (0, 0, 0),
                         pipeline_mode=pl.Buffered(1)),
        ],
        out_specs=pl.BlockSpec((tile, 1, D), lambda i: (i, 0, 0)),
        scratch_shapes=[
            pltpu.SMEM((tile,), jnp.int32),
            pltpu.SemaphoreType.DMA((tile // _CHUNK,)),
        ],
        compiler_params=pltpu.CompilerParams(
            dimension_semantics=("arbitrary",),
            vmem_limit_bytes=_VMEM_LIMIT,
        ),
    )(ids3, table3)


@jax.jit
def _emoji_gather(emojis, table):
    B, S = emojis.shape
    V, D = table.shape
    n = B * S

    ids = emojis.reshape(-1).astype(jnp.int32)
    tile = _TILE if n >= _TILE else pl.cdiv(n, _CHUNK) * _CHUNK
    n_pad = pl.cdiv(n, tile) * tile
    n_tiles = n_pad // tile
    if n_pad != n:
        ids = jnp.pad(ids, (0, n_pad - n))
    ids3 = ids.reshape(n_tiles, 1, tile)
    table3 = table.astype(jnp.float32).reshape(V, 1, D)

    ndev = len(jax.devices())
    if ndev > 1 and n_pad == n and n_tiles % ndev == 0 and B % ndev == 0:
        mesh = jax.make_mesh(
            (ndev,), ("c",),
            axis_types=(jax.sharding.AxisType.Explicit,),
        )
        ids3 = jax.reshard(ids3, jax.NamedSharding(mesh, P("c", None, None)))
        table3 = jax.reshard(table3, jax.NamedSharding(mesh, P()))

        def _local(i3, t3):
            return _gather_tiles(i3, t3).reshape(B // ndev, S, D)

        return jax.shard_map(
            _local, mesh=mesh, in_specs=(P("c"), P()), out_specs=P("c"),
            check_vma=False,
        )(ids3, table3)

    out = _gather_tiles(ids3, table3)
    if n_pad != n:
        out = out[:n]
    return out.reshape(B, S, D)


def kernel(emojis, table):
    return _emoji_gather(emojis, table)
